# baseline (speedup 1.0000x reference)
import jax
import jax.numpy as jnp
from jax.experimental import pallas as pl

_K = 8

_SORT8 = [
    (0, 1), (2, 3), (4, 5), (6, 7),
    (0, 2), (1, 3), (4, 6), (5, 7),
    (1, 2), (5, 6),
    (0, 4), (1, 5), (2, 6), (3, 7),
    (2, 4), (3, 5),
    (1, 2), (3, 4), (5, 6),
]

_BITONIC8 = [
    (0, 4), (1, 5), (2, 6), (3, 7),
    (0, 2), (1, 3), (4, 6), (5, 7),
    (0, 1), (2, 3), (4, 5), (6, 7),
]


def _cx(a, i, j):
    hi = jnp.maximum(a[i], a[j])
    lo = jnp.minimum(a[i], a[j])
    a[i] = hi
    a[j] = lo


def _merge8(acc, s):
    m = [jnp.maximum(acc[i], s[_K - 1 - i]) for i in range(_K)]
    for (i, j) in _BITONIC8:
        _cx(m, i, j)
    return m


def _topk_kernel(x_ref, o_ref):
    t = x_ref.shape[1]
    chunks = t // 64

    def chunk_sorted(base):
        s = [x_ref[0, pl.ds(base + _K * j, _K), :] for j in range(_K)]
        for (i, j) in _SORT8:
            _cx(s, i, j)
        return s

    def body(m, accs):
        acc0, acc1 = accs
        base = m * 128
        acc0 = _merge8(list(acc0), chunk_sorted(base))
        acc1 = _merge8(list(acc1), chunk_sorted(base + 64))
        return (tuple(acc0), tuple(acc1))

    init1 = tuple(
        jnp.full((_K, x_ref.shape[2]), -jnp.inf, dtype=x_ref.dtype)
        for _ in range(_K)
    )
    acc0, acc1 = jax.lax.fori_loop(0, chunks // 2, body, (init1, init1),
                                   unroll=2)
    a = _merge8(list(acc0), list(acc1))
    h = _K // 2
    while h >= 1:
        top = [v[:h, :] for v in a]
        bot = [v[h:2 * h, :] for v in a]
        a = _merge8(top, bot)
        h //= 2
    for i in range(_K):
        o_ref[0, i, :] = a[i][0]


def kernel(inputs):
    b, t, c = inputs.shape
    cb = 256
    out = pl.pallas_call(
        _topk_kernel,
        grid=(b, c // cb),
        in_specs=[pl.BlockSpec((1, t, cb), lambda i, j: (i, 0, j))],
        out_specs=pl.BlockSpec((1, _K, cb), lambda i, j: (i, 0, j)),
        out_shape=jax.ShapeDtypeStruct((b, _K, c), inputs.dtype),
    )(inputs)
    return out.reshape(b, _K * c)


# speedup vs baseline: 1.1419x; 1.1419x over previous
import jax
import jax.numpy as jnp
from jax.experimental import pallas as pl

_K = 8

_SORT8 = [
    (0, 1), (2, 3), (4, 5), (6, 7),
    (0, 2), (1, 3), (4, 6), (5, 7),
    (1, 2), (5, 6),
    (0, 4), (1, 5), (2, 6), (3, 7),
    (2, 4), (3, 5),
    (1, 2), (3, 4), (5, 6),
]

_BITONIC8 = [
    (0, 4), (1, 5), (2, 6), (3, 7),
    (0, 2), (1, 3), (4, 6), (5, 7),
    (0, 1), (2, 3), (4, 5), (6, 7),
]


def _cx(a, i, j):
    hi = jnp.maximum(a[i], a[j])
    lo = jnp.minimum(a[i], a[j])
    a[i] = hi
    a[j] = lo


def _merge8(acc, s):
    m = [jnp.maximum(acc[i], s[_K - 1 - i]) for i in range(_K)]
    for (i, j) in _BITONIC8:
        _cx(m, i, j)
    return m


def _topk_kernel(x_ref, o_ref):
    t = x_ref.shape[1]
    chunks = t // 64

    def chunk_sorted(base):
        s = [x_ref[0, pl.ds(base + _K * j, _K), :] for j in range(_K)]
        for (i, j) in _SORT8:
            _cx(s, i, j)
        return s

    def body(m, accs):
        acc0, acc1 = accs
        base = m * 128
        acc0 = _merge8(list(acc0), chunk_sorted(base))
        acc1 = _merge8(list(acc1), chunk_sorted(base + 64))
        return (tuple(acc0), tuple(acc1))

    init1 = tuple(
        jnp.full((_K, x_ref.shape[2]), -jnp.inf, dtype=x_ref.dtype)
        for _ in range(_K)
    )
    acc0, acc1 = jax.lax.fori_loop(0, chunks // 2, body, (init1, init1),
                                   unroll=4)
    a = _merge8(list(acc0), list(acc1))
    h = _K // 2
    while h >= 1:
        top = [v[:h, :] for v in a]
        bot = [v[h:2 * h, :] for v in a]
        a = _merge8(top, bot)
        h //= 2
    for i in range(_K):
        o_ref[0, i, :] = a[i][0]


def kernel(inputs):
    b, t, c = inputs.shape
    cb = 512
    out = pl.pallas_call(
        _topk_kernel,
        grid=(b, c // cb),
        in_specs=[pl.BlockSpec((1, t, cb), lambda i, j: (i, 0, j))],
        out_specs=pl.BlockSpec((1, _K, cb), lambda i, j: (i, 0, j)),
        out_shape=jax.ShapeDtypeStruct((b, _K, c), inputs.dtype),
    )(inputs)
    return out.reshape(b, _K * c)


# speedup vs baseline: 1.1432x; 1.0012x over previous
import jax
import jax.numpy as jnp
from jax.experimental import pallas as pl

_K = 8

_SORT8 = [
    (0, 1), (2, 3), (4, 5), (6, 7),
    (0, 2), (1, 3), (4, 6), (5, 7),
    (1, 2), (5, 6),
    (0, 4), (1, 5), (2, 6), (3, 7),
    (2, 4), (3, 5),
    (1, 2), (3, 4), (5, 6),
]

_BITONIC8 = [
    (0, 4), (1, 5), (2, 6), (3, 7),
    (0, 2), (1, 3), (4, 6), (5, 7),
    (0, 1), (2, 3), (4, 5), (6, 7),
]


def _cx(a, i, j):
    hi = jnp.maximum(a[i], a[j])
    lo = jnp.minimum(a[i], a[j])
    a[i] = hi
    a[j] = lo


def _merge8(acc, s):
    m = [jnp.maximum(acc[i], s[_K - 1 - i]) for i in range(_K)]
    for (i, j) in _BITONIC8:
        _cx(m, i, j)
    return m


def _topk_kernel(x_ref, o_ref):
    t = x_ref.shape[1]
    chunks = t // 64

    def chunk_sorted(base):
        s = [x_ref[0, pl.ds(base + _K * j, _K), :] for j in range(_K)]
        for (i, j) in _SORT8:
            _cx(s, i, j)
        return s

    def body(m, acc):
        return tuple(_merge8(list(acc), chunk_sorted(m * 64)))

    init1 = tuple(
        jnp.full((_K, x_ref.shape[2]), -jnp.inf, dtype=x_ref.dtype)
        for _ in range(_K)
    )
    a = list(jax.lax.fori_loop(0, chunks, body, init1, unroll=4))
    h = _K // 2
    while h >= 1:
        top = [v[:h, :] for v in a]
        bot = [v[h:2 * h, :] for v in a]
        a = _merge8(top, bot)
        h //= 2
    for i in range(_K):
        o_ref[0, i, :] = a[i][0]


def kernel(inputs):
    b, t, c = inputs.shape
    cb = 512
    out = pl.pallas_call(
        _topk_kernel,
        grid=(b, c // cb),
        in_specs=[pl.BlockSpec((1, t, cb), lambda i, j: (i, 0, j))],
        out_specs=pl.BlockSpec((1, _K, cb), lambda i, j: (i, 0, j)),
        out_shape=jax.ShapeDtypeStruct((b, _K, c), inputs.dtype),
    )(inputs)
    return out.reshape(b, _K * c)


# speedup vs baseline: 1.1799x; 1.0320x over previous
import jax
import jax.numpy as jnp
from jax.experimental import pallas as pl

_K = 8

_SORT8 = [
    (0, 1), (2, 3), (4, 5), (6, 7),
    (0, 2), (1, 3), (4, 6), (5, 7),
    (1, 2), (5, 6),
    (0, 4), (1, 5), (2, 6), (3, 7),
    (2, 4), (3, 5),
    (1, 2), (3, 4), (5, 6),
]

_BITONIC8 = [
    (0, 4), (1, 5), (2, 6), (3, 7),
    (0, 2), (1, 3), (4, 6), (5, 7),
    (0, 1), (2, 3), (4, 5), (6, 7),
]


def _cx(a, i, j):
    hi = jnp.maximum(a[i], a[j])
    lo = jnp.minimum(a[i], a[j])
    a[i] = hi
    a[j] = lo


def _merge8(acc, s):
    m = [jnp.maximum(acc[i], s[_K - 1 - i]) for i in range(_K)]
    for (i, j) in _BITONIC8:
        _cx(m, i, j)
    return m


def _topk_kernel(x_ref, o_ref):
    t = x_ref.shape[1]
    chunks = t // 64

    def chunk_sorted(base):
        s = [x_ref[0, pl.ds(base + _K * j, _K), :] for j in range(_K)]
        for (i, j) in _SORT8:
            _cx(s, i, j)
        return s

    a = chunk_sorted(0)
    for m in range(1, chunks):
        a = _merge8(a, chunk_sorted(m * 64))
    h = _K // 2
    while h >= 1:
        top = [v[:h, :] for v in a]
        bot = [v[h:2 * h, :] for v in a]
        a = _merge8(top, bot)
        h //= 2
    for i in range(_K):
        o_ref[0, i, :] = a[i][0]


def kernel(inputs):
    b, t, c = inputs.shape
    cb = 512
    out = pl.pallas_call(
        _topk_kernel,
        grid=(b, c // cb),
        in_specs=[pl.BlockSpec((1, t, cb), lambda i, j: (i, 0, j))],
        out_specs=pl.BlockSpec((1, _K, cb), lambda i, j: (i, 0, j)),
        out_shape=jax.ShapeDtypeStruct((b, _K, c), inputs.dtype),
    )(inputs)
    return out.reshape(b, _K * c)
